# Initial kernel scaffold; baseline (speedup 1.0000x reference)
import jax
import jax.numpy as jnp
from jax.experimental import pallas as pl


def kernel(actions, psi_prime, knn_indices, demands, coords, capacity, Wq_w, Wq_b, lambda_param, mu_param):
    raise NotImplementedError("write your pallas kernel here")


# speedup vs baseline: 251.8377x; 251.8377x over previous
import functools

import jax
import jax.numpy as jnp
from jax import lax
from jax.experimental import pallas as pl
from jax.experimental.pallas import tpu as pltpu
from jax.experimental.pallas import tpu_sc as plsc

B, T, N1, K = 256, 64, 1000, 16
NPAD = 1024
NC, NS = 2, 16
NW = NC * NS
BPW = B // NW
LANES = 16
NCHUNK = NPAD // LANES


def _sc_interference_body(x_hbm, y_hbm, knn_hbm, out_hbm, x_v, y_v, knn_v, out_v):
    wid = lax.axis_index("s") * NC + lax.axis_index("c")

    def batch_body(i, carry):
        b = wid * BPW + i
        pltpu.sync_copy(x_hbm.at[b], x_v)
        pltpu.sync_copy(y_hbm.at[b], y_v)
        pltpu.sync_copy(knn_hbm.at[b], knn_v)

        def chunk_body(c, carry2):
            base = c * LANES
            gx = jnp.zeros((LANES,), jnp.float32)
            gy = jnp.zeros((LANES,), jnp.float32)
            for k in range(K):
                idx = knn_v[k, pl.ds(base, LANES)]
                gx = gx + plsc.load_gather(x_v, [idx])
                gy = gy + plsc.load_gather(y_v, [idx])
            ox = x_v[pl.ds(base, LANES)]
            oy = y_v[pl.ds(base, LANES)]
            out_v[pl.ds(base, LANES)] = ox * gx + oy * gy
            return carry2

        lax.fori_loop(0, NCHUNK, chunk_body, 0)
        pltpu.sync_copy(out_v, out_hbm.at[b])
        return carry

    lax.fori_loop(0, BPW, batch_body, 0)


def _sc_interference(x, y, knn_t):
    mesh = plsc.VectorSubcoreMesh(core_axis_name="c", subcore_axis_name="s")
    return pl.kernel(
        _sc_interference_body,
        out_type=jax.ShapeDtypeStruct((B, NPAD), jnp.float32),
        mesh=mesh,
        compiler_params=pltpu.CompilerParams(needs_layout_passes=False),
        scratch_types=[
            pltpu.VMEM((NPAD,), jnp.float32),
            pltpu.VMEM((NPAD,), jnp.float32),
            pltpu.VMEM((K, NPAD), jnp.int32),
            pltpu.VMEM((NPAD,), jnp.float32),
        ],
    )(x, y, knn_t)


BB = 8


def _tc_body(act_ref, x_ref, y_ref, cx_ref, cy_ref, dem_ref, intf_ref,
             cap_ref, par_ref, lp_ref, ent_ref):
    act = act_ref[...]
    x = x_ref[...]
    y = y_ref[...]
    cx = cx_ref[...]
    cy = cy_ref[...]
    dem = dem_ref[...]
    intf = intf_ref[...]
    cap = cap_ref[:, 0:1]
    par = par_ref[...]

    def w(i):
        return par[0, i]

    n_io = lax.broadcasted_iota(jnp.int32, (1, 1, NPAD), 2)
    t_io = lax.broadcasted_iota(jnp.int32, (1, T, 1), 1)

    eq = act[:, :, None] == n_io
    cur_idx = jnp.concatenate(
        [jnp.zeros((BB, 1), jnp.int32), act[:, : T - 1]], axis=1)
    cq = cur_idx[:, :, None] == n_io

    fv = jnp.min(jnp.where(eq, t_io, T), axis=1)
    visited = (t_io > fv[:, None, :]) & (n_io > 0)

    dem_t = jnp.sum(jnp.where(eq, dem[:, None, :], 0.0), axis=-1)
    depot_s = act == 0
    d_s = jnp.where(depot_s, 0.0, dem_t)
    incl = depot_s.astype(jnp.int32)
    for sh in (1, 2, 4, 8, 16, 32):
        incl = incl + jnp.concatenate(
            [jnp.zeros((BB, sh), jnp.int32), incl[:, : T - sh]], axis=1)
    dex = incl - depot_s.astype(jnp.int32)
    s_io2 = lax.broadcasted_iota(jnp.int32, (1, 1, T), 2)
    t_io2 = lax.broadcasted_iota(jnp.int32, (1, T, 1), 1)
    seg = (s_io2 < t_io2) & (incl[:, None, :] == dex[:, :, None])
    used = jnp.sum(jnp.where(seg, d_s[:, None, :], 0.0), axis=-1)

    remaining = cap - used
    cap_norm = remaining / jnp.maximum(cap, 1e-8)

    psx = jnp.sum(jnp.where(cq, x[:, None, :], 0.0), axis=-1)
    psy = jnp.sum(jnp.where(cq, y[:, None, :], 0.0), axis=-1)
    ccx = jnp.sum(jnp.where(cq, cx[:, None, :], 0.0), axis=-1)
    ccy = jnp.sum(jnp.where(cq, cy[:, None, :], 0.0), axis=-1)
    at_dep = jnp.concatenate(
        [jnp.ones((BB, 1), jnp.int32),
         depot_s[:, : T - 1].astype(jnp.int32)], axis=1) != 0
    psx = jnp.where(at_dep, 0.0, psx)
    psy = jnp.where(at_dep, 0.0, psy)

    t_norm = (lax.broadcasted_iota(jnp.int32, (BB, T), 1).astype(jnp.float32)
              / float(N1 - 1))

    qx = (psx * w(0) + psy * w(2) + cap_norm * w(4) + t_norm * w(6)
          + ccx * w(8) + ccy * w(10) + w(12))
    qy = (psx * w(1) + psy * w(3) + cap_norm * w(5) + t_norm * w(7)
          + ccx * w(9) + ccy * w(11) + w(13))
    lam = w(14)
    mu = w(15)

    exceeds = dem[:, None, :] > remaining[:, :, None]
    mask_c = visited | exceeds
    valid_cust = (n_io >= 1) & (n_io < N1)
    has_cust = jnp.any((~mask_c) & valid_cust, axis=-1)
    m0f = (at_dep & has_cust).astype(jnp.float32)
    maskf = jnp.where(mask_c, 1.0, 0.0)
    maskf = jnp.where(n_io == 0, m0f[:, :, None], maskf)
    final_mask = (maskf > 0.5) | (n_io >= N1)

    cs = qx[:, :, None] * x[:, None, :] + qy[:, :, None] * y[:, None, :]
    dx = cx[:, None, :] - ccx[:, :, None]
    dy = cy[:, None, :] - ccy[:, :, None]
    dist = jnp.sqrt(dx * dx + dy * dy + 1e-12)
    sc = cs + lam * intf[:, None, :] - mu * dist
    sc = jnp.where(final_mask, -1e9, sc)

    m = jnp.max(sc, axis=-1, keepdims=True)
    e = jnp.exp(sc - m)
    z = jnp.sum(e, axis=-1, keepdims=True)
    logz = (m + jnp.log(z))[..., 0]
    lp_act = jnp.sum(jnp.where(eq, sc, 0.0), axis=-1)
    psum = jnp.sum(e * sc, axis=-1) / z[..., 0]
    ent = jnp.mean(logz - psum, axis=1)

    lp_ref[...] = lp_act - logz
    ent_ref[...] = jnp.broadcast_to(ent[:, None], (BB, 128))


def _tc_decode(act, x, y, cx, cy, dem, intf, cap128, params):
    grid = (B // BB,)
    row = lambda i: (i, 0)
    return pl.pallas_call(
        _tc_body,
        grid=grid,
        in_specs=[
            pl.BlockSpec((BB, T), row),
            pl.BlockSpec((BB, NPAD), row),
            pl.BlockSpec((BB, NPAD), row),
            pl.BlockSpec((BB, NPAD), row),
            pl.BlockSpec((BB, NPAD), row),
            pl.BlockSpec((BB, NPAD), row),
            pl.BlockSpec((BB, NPAD), row),
            pl.BlockSpec((BB, 128), row),
            pl.BlockSpec((1, 16), lambda i: (0, 0)),
        ],
        out_specs=[
            pl.BlockSpec((BB, T), row),
            pl.BlockSpec((BB, 128), row),
        ],
        out_shape=[
            jax.ShapeDtypeStruct((B, T), jnp.float32),
            jax.ShapeDtypeStruct((B, 128), jnp.float32),
        ],
    )(act, x, y, cx, cy, dem, intf, cap128, params)


def kernel(actions, psi_prime, knn_indices, demands, coords, capacity,
           Wq_w, Wq_b, lambda_param, mu_param):
    pad = [(0, 0), (0, NPAD - N1)]
    x = jnp.pad(psi_prime[:, :, 0], pad)
    y = jnp.pad(psi_prime[:, :, 1], pad)
    cx = jnp.pad(coords[:, :, 0], pad)
    cy = jnp.pad(coords[:, :, 1], pad)
    dem = jnp.pad(demands, pad)
    knn_t = jnp.pad(jnp.transpose(knn_indices, (0, 2, 1)),
                    [(0, 0), (0, 0), (0, NPAD - N1)])
    cap128 = jnp.broadcast_to(capacity[:, None], (B, 128))
    params = jnp.concatenate(
        [Wq_w.reshape(-1), Wq_b.reshape(-1),
         lambda_param.reshape(1), mu_param.reshape(1)]).reshape(1, 16)

    intf = _sc_interference(x, y, knn_t.astype(jnp.int32))
    lp, ent128 = _tc_decode(actions.astype(jnp.int32), x, y, cx, cy, dem,
                            intf, cap128, params)
    return (lp, ent128[:, 0])
